# Initial kernel scaffold; baseline (speedup 1.0000x reference)
import jax
import jax.numpy as jnp
from jax.experimental import pallas as pl


def kernel(x, pos, TE, PE):
    raise NotImplementedError("write your pallas kernel here")


# speedup vs baseline: 1.3597x; 1.3597x over previous
import functools

import jax
import jax.numpy as jnp
from jax import lax
from jax.experimental import pallas as pl
from jax.experimental.pallas import tpu as pltpu
from jax.experimental.pallas import tpu_sc as plsc

_NC = 2
_NS = 16
_L = 16


def kernel(x, pos, TE, PE):
    B, S = x.shape
    D = TE.shape[1]
    N = B * S
    NW = _NC * _NS
    n_per_w = N // NW
    C = 64
    n_chunks = n_per_w // C
    vecs_per_row = D // _L

    xf = x.reshape(N).astype(jnp.int32)
    pf = pos.reshape(N).astype(jnp.int32)

    mesh = plsc.VectorSubcoreMesh(core_axis_name="c", subcore_axis_name="s")

    @functools.partial(
        pl.kernel,
        mesh=mesh,
        out_type=jax.ShapeDtypeStruct((N, D), jnp.float32),
        scratch_types=[
            pltpu.VMEM((C,), jnp.int32),
            pltpu.VMEM((C,), jnp.int32),
            pltpu.VMEM((C, D), jnp.float32),
            pltpu.VMEM((C, D), jnp.float32),
            pltpu.SemaphoreType.DMA,
            pltpu.SemaphoreType.DMA,
        ],
    )
    def emb(x_hbm, p_hbm, te_hbm, pe_hbm, out_hbm, xi_v, pi_v, tok_v, pos_v,
            sem_t, sem_p):
        wid = lax.axis_index("s") * _NC + lax.axis_index("c")
        base = wid * n_per_w

        def chunk_body(c, _):
            off = base + c * C
            pltpu.sync_copy(x_hbm.at[pl.ds(off, C)], xi_v)
            pltpu.sync_copy(p_hbm.at[pl.ds(off, C)], pi_v)
            cp_t = pltpu.async_copy(te_hbm.at[xi_v], tok_v, sem_t)
            cp_p = pltpu.async_copy(pe_hbm.at[pi_v], pos_v, sem_p)
            cp_t.wait()
            cp_p.wait()

            def add_row(r, _):
                for j in range(vecs_per_row):
                    sl = pl.ds(j * _L, _L)
                    plsc.addupdate(tok_v.at[r, sl], pos_v[r, sl])
                return 0

            lax.fori_loop(0, C, add_row, 0)
            pltpu.sync_copy(tok_v, out_hbm.at[pl.ds(off, C)])
            return 0

        lax.fori_loop(0, n_chunks, chunk_body, 0)

    out = emb(xf, pf, TE, PE)
    return out.reshape(B, S, D)
